# baseline (speedup 1.0000x reference)
import jax
import jax.numpy as jnp
from jax import lax
from jax.experimental import pallas as pl
from jax.experimental.pallas import tpu as pltpu
from jax.experimental.pallas import tpu_sc as plsc

BATCH = 16384
D = 32
PACK = 4
W128 = 128
L = 16
NC = 2
NS = 16
NW = NC * NS
BPW = BATCH // NW
CHUNK = 128
NCHUNK = BPW // CHUNK
GPC = CHUNK // L


NT = 1000000 // W128 + 1
TPW = NT // NW + 1


def _tp_body(t3_hbm, out_hbm, vin_v, vout_v):
    c = lax.axis_index("c")
    s = lax.axis_index("s")
    wid = s * NC + c
    iota = lax.iota(jnp.int32, L)

    def col(k, carry):
        t = k * NW + wid

        @pl.when(t < NT - 1)
        def _full():
            for a in range(PACK):
                pltpu.sync_copy(t3_hbm.at[a, :, pl.ds(t * W128, W128)],
                                vin_v.at[pl.ds(a * 8, 8), :])

            def qrow(q, c2):
                base = 4 * q
                for h in range(2):
                    rows = h * L + iota
                    for sft in range(PACK):
                        piece = plsc.load_gather(
                            vin_v, [rows, jnp.full((L,), 0, jnp.int32) + base + sft])
                        vout_v[q, pl.ds(sft * D + h * L, L)] = piece
                return c2
            lax.fori_loop(0, D, qrow, 0)
            pltpu.sync_copy(vout_v, out_hbm.at[pl.ds(t * D, D), :])

        @pl.when(t == NT - 1)
        def _tail():
            for a in range(PACK):
                pltpu.sync_copy(t3_hbm.at[a, :, pl.ds(t * W128, 64)],
                                vin_v.at[pl.ds(a * 8, 8), pl.ds(0, 64)])

            def qrow(q, c2):
                base = 4 * q
                for h in range(2):
                    rows = h * L + iota
                    for sft in range(PACK):
                        piece = plsc.load_gather(
                            vin_v, [rows, jnp.full((L,), 0, jnp.int32) + base + sft])
                        vout_v[q, pl.ds(sft * D + h * L, L)] = piece
                return c2
            lax.fori_loop(0, 16, qrow, 0)
            pltpu.sync_copy(vout_v.at[pl.ds(0, 16), :],
                            out_hbm.at[pl.ds(t * D, 16), :])
        return carry

    lax.fori_loop(0, TPW, col, 0)


def _mf_body(user_hbm, item_hbm, uw_hbm, iw_hbm, out_hbm,
             uidx_v, iidx_v, uq_v, iq_v, urows_v, irows_v, out_v, sem):
    c = lax.axis_index("c")
    s = lax.axis_index("s")
    wid = s * NC + c

    pltpu.sync_copy(user_hbm.at[pl.ds(wid * NCHUNK, NCHUNK)], uidx_v)
    pltpu.sync_copy(item_hbm.at[pl.ds(wid * NCHUNK, NCHUNK)], iidx_v)

    iota = lax.iota(jnp.int32, L)

    def rnd(j, carry):
        def qgrp(i, c2):
            uvec = uidx_v[j, pl.ds(i * L, L)]
            ivec = iidx_v[j, pl.ds(i * L, L)]
            uq_v[pl.ds(i * L, L)] = lax.shift_right_logical(uvec, 2)
            iq_v[pl.ds(i * L, L)] = lax.shift_right_logical(ivec, 2)
            return c2
        lax.fori_loop(0, GPC, qgrp, 0)

        cp1 = pltpu.async_copy(uw_hbm.at[uq_v], urows_v, sem)
        cp2 = pltpu.async_copy(iw_hbm.at[iq_v], irows_v, sem)
        cp1.wait()
        cp2.wait()

        def grp(i, c2):
            rows = i * L + iota
            uvec = uidx_v[j, pl.ds(i * L, L)]
            ivec = iidx_v[j, pl.ds(i * L, L)]
            uoff = lax.shift_left(jnp.bitwise_and(uvec, 3), 5)
            ioff = lax.shift_left(jnp.bitwise_and(ivec, 3), 5)
            acc = jnp.zeros((L,), jnp.float32)
            for d in range(D):
                cu = plsc.load_gather(urows_v, [rows, uoff + d])
                cv = plsc.load_gather(irows_v, [rows, ioff + d])
                acc = acc + cu * cv
            e = jnp.exp(-jnp.abs(acc))
            p = 1.0 / (1.0 + e)
            out_v[pl.ds(j * CHUNK + i * L, L)] = jnp.where(acc >= 0, p, 1.0 - p)
            return c2
        lax.fori_loop(0, GPC, grp, 0)
        return carry

    lax.fori_loop(0, NCHUNK, rnd, 0)
    pltpu.sync_copy(out_v, out_hbm.at[pl.ds(wid * BPW, BPW)])


@jax.jit
def kernel(user, item, user_embed_w, item_embed_w):
    mesh = plsc.VectorSubcoreMesh(core_axis_name="c", subcore_axis_name="s",
                                  num_cores=NC, num_subcores=NS)
    mf = pl.kernel(
        _mf_body,
        out_type=jax.ShapeDtypeStruct((BATCH,), jnp.float32),
        mesh=mesh,
        scratch_types=[
            pltpu.VMEM((NCHUNK, CHUNK), jnp.int32),
            pltpu.VMEM((NCHUNK, CHUNK), jnp.int32),
            pltpu.VMEM((CHUNK,), jnp.int32),
            pltpu.VMEM((CHUNK,), jnp.int32),
            pltpu.VMEM((CHUNK, W128), jnp.float32),
            pltpu.VMEM((CHUNK, W128), jnp.float32),
            pltpu.VMEM((BPW,), jnp.float32),
            pltpu.SemaphoreType.DMA,
        ],
        compiler_params=pltpu.CompilerParams(
            needs_layout_passes=False, use_tc_tiling_on_sc=True),
    )
    tp = pl.kernel(
        _tp_body,
        out_type=jax.ShapeDtypeStruct((1000000 * D // W128, W128),
                                      jnp.float32),
        mesh=mesh,
        scratch_types=[
            pltpu.VMEM((D, W128), jnp.float32),
            pltpu.VMEM((D, W128), jnp.float32),
        ],
        compiler_params=pltpu.CompilerParams(
            needs_layout_passes=False, use_tc_tiling_on_sc=True),
    )
    user2d = user.reshape(NW * NCHUNK, CHUNK)
    item2d = item.reshape(NW * NCHUNK, CHUNK)
    uw3 = user_embed_w.T.reshape(PACK, 8, 1000000)
    iw3 = item_embed_w.T.reshape(PACK, 8, 1000000)
    uw_packed = tp(uw3)
    iw_packed = tp(iw3)
    return mf(user2d, item2d, uw_packed, iw_packed)


# speedup vs baseline: 1.7803x; 1.7803x over previous
import jax
import jax.numpy as jnp
from jax import lax
from jax.experimental import pallas as pl
from jax.experimental.pallas import tpu as pltpu
from jax.experimental.pallas import tpu_sc as plsc

BATCH = 16384
D = 32
PACK = 4
W128 = 128
L = 16
NC = 2
NS = 16
NW = NC * NS
BPW = BATCH // NW
CHUNK = 128
NCHUNK = BPW // CHUNK
GPC = CHUNK // L


ROWS_SC = 512
FULL_SC = 1000000 // ROWS_SC
TAIL_ROWS = 1000000 - FULL_SC * ROWS_SC
TAIL_WID = FULL_SC % NW
KMAX = FULL_SC // NW + 1


def _tp_body(t3_hbm, out_hbm, vin0, vin1, vout0, vout1, vtail,
             si0, si1, so0, so1):
    c = lax.axis_index("c")
    s = lax.axis_index("s")
    wid = s * NC + c
    iota = lax.iota(jnp.int32, L)
    zero = jnp.full((L,), 0, jnp.int32)

    def fire_in(sc, vin, si):
        for a in range(PACK):
            pltpu.async_copy(t3_hbm.at[a, :, pl.ds(sc * ROWS_SC, ROWS_SC)],
                             vin.at[pl.ds(a * 8, 8), :], si)

    def drain_in(sc, vin, si):
        for a in range(PACK):
            pltpu.make_async_copy(
                t3_hbm.at[a, :, pl.ds(sc * ROWS_SC, ROWS_SC)],
                vin.at[pl.ds(a * 8, 8), :], si).wait()

    def compute(vin, vout, nq):
        def qrow(q, c2):
            b4 = 4 * q
            for h in range(2):
                rows = h * L + iota
                for sft in range(PACK):
                    piece = plsc.load_gather(vin, [rows, zero + b4 + sft])
                    vout[q, pl.ds(sft * D + h * L, L)] = piece
            return c2
        lax.fori_loop(0, nq, qrow, 0)

    def step(k, vin, vout, si, so):
        sc = k * NW + wid
        nsc = sc + NW
        nvin, nsi = (vin1, si1) if vin is vin0 else (vin0, si0)

        @pl.when(nsc < FULL_SC)
        def _fire_next():
            fire_in(nsc, nvin, nsi)

        @pl.when(sc < FULL_SC)
        def _do():
            drain_in(sc, vin, si)

            @pl.when(k >= 2)
            def _drain_old_out():
                pltpu.make_async_copy(
                    vout, out_hbm.at[pl.ds(sc * 128, 128), :], so).wait()

            compute(vin, vout, 128)
            pltpu.async_copy(vout, out_hbm.at[pl.ds(sc * 128, 128), :], so)

    fire_in(wid, vin0, si0)

    def body(k, carry):
        @pl.when(k % 2 == 0)
        def _even():
            step(k, vin0, vout0, si0, so0)

        @pl.when(k % 2 == 1)
        def _odd():
            step(k, vin1, vout1, si1, so1)
        return carry

    lax.fori_loop(0, KMAX, body, 0)

    pltpu.make_async_copy(vout0, out_hbm.at[pl.ds(0, 128), :], so0).wait()
    pltpu.make_async_copy(vout1, out_hbm.at[pl.ds(0, 128), :], so1).wait()

    @pl.when(wid == TAIL_WID)
    def _tail():
        toff = (wid - TAIL_WID + 1) * (FULL_SC * ROWS_SC)
        for a in range(PACK):
            pltpu.sync_copy(
                t3_hbm.at[a, :, pl.ds(toff, TAIL_ROWS)],
                vtail.at[pl.ds(a * 8, 8), pl.ds(0, TAIL_ROWS)])
        compute(vtail, vout0, TAIL_ROWS // PACK)
        pltpu.sync_copy(vout0.at[pl.ds(0, TAIL_ROWS // PACK), :],
                        out_hbm.at[pl.ds(FULL_SC * 128, TAIL_ROWS // PACK), :])


def _mf_body(user_hbm, item_hbm, uw_hbm, iw_hbm, out_hbm,
             uidx_v, iidx_v, uq_v, iq_v, urows_v, irows_v, out_v, sem):
    c = lax.axis_index("c")
    s = lax.axis_index("s")
    wid = s * NC + c

    pltpu.sync_copy(user_hbm.at[pl.ds(wid * NCHUNK, NCHUNK)], uidx_v)
    pltpu.sync_copy(item_hbm.at[pl.ds(wid * NCHUNK, NCHUNK)], iidx_v)

    iota = lax.iota(jnp.int32, L)

    def rnd(j, carry):
        def qgrp(i, c2):
            uvec = uidx_v[j, pl.ds(i * L, L)]
            ivec = iidx_v[j, pl.ds(i * L, L)]
            uq_v[pl.ds(i * L, L)] = lax.shift_right_logical(uvec, 2)
            iq_v[pl.ds(i * L, L)] = lax.shift_right_logical(ivec, 2)
            return c2
        lax.fori_loop(0, GPC, qgrp, 0)

        cp1 = pltpu.async_copy(uw_hbm.at[uq_v], urows_v, sem)
        cp2 = pltpu.async_copy(iw_hbm.at[iq_v], irows_v, sem)
        cp1.wait()
        cp2.wait()

        def grp(i, c2):
            rows = i * L + iota
            uvec = uidx_v[j, pl.ds(i * L, L)]
            ivec = iidx_v[j, pl.ds(i * L, L)]
            uoff = lax.shift_left(jnp.bitwise_and(uvec, 3), 5)
            ioff = lax.shift_left(jnp.bitwise_and(ivec, 3), 5)
            acc = jnp.zeros((L,), jnp.float32)
            for d in range(D):
                cu = plsc.load_gather(urows_v, [rows, uoff + d])
                cv = plsc.load_gather(irows_v, [rows, ioff + d])
                acc = acc + cu * cv
            e = jnp.exp(-jnp.abs(acc))
            p = 1.0 / (1.0 + e)
            out_v[pl.ds(j * CHUNK + i * L, L)] = jnp.where(acc >= 0, p, 1.0 - p)
            return c2
        lax.fori_loop(0, GPC, grp, 0)
        return carry

    lax.fori_loop(0, NCHUNK, rnd, 0)
    pltpu.sync_copy(out_v, out_hbm.at[pl.ds(wid * BPW, BPW)])


@jax.jit
def kernel(user, item, user_embed_w, item_embed_w):
    mesh = plsc.VectorSubcoreMesh(core_axis_name="c", subcore_axis_name="s",
                                  num_cores=NC, num_subcores=NS)
    mf = pl.kernel(
        _mf_body,
        out_type=jax.ShapeDtypeStruct((BATCH,), jnp.float32),
        mesh=mesh,
        scratch_types=[
            pltpu.VMEM((NCHUNK, CHUNK), jnp.int32),
            pltpu.VMEM((NCHUNK, CHUNK), jnp.int32),
            pltpu.VMEM((CHUNK,), jnp.int32),
            pltpu.VMEM((CHUNK,), jnp.int32),
            pltpu.VMEM((CHUNK, W128), jnp.float32),
            pltpu.VMEM((CHUNK, W128), jnp.float32),
            pltpu.VMEM((BPW,), jnp.float32),
            pltpu.SemaphoreType.DMA,
        ],
        compiler_params=pltpu.CompilerParams(
            needs_layout_passes=False, use_tc_tiling_on_sc=True),
    )
    tp = pl.kernel(
        _tp_body,
        out_type=jax.ShapeDtypeStruct((1000000 * D // W128, W128),
                                      jnp.float32),
        mesh=mesh,
        scratch_types=[
            pltpu.VMEM((D, ROWS_SC), jnp.float32),
            pltpu.VMEM((D, ROWS_SC), jnp.float32),
            pltpu.VMEM((W128, W128), jnp.float32),
            pltpu.VMEM((W128, W128), jnp.float32),
            pltpu.VMEM((D, W128), jnp.float32),
            pltpu.SemaphoreType.DMA,
            pltpu.SemaphoreType.DMA,
            pltpu.SemaphoreType.DMA,
            pltpu.SemaphoreType.DMA,
        ],
        compiler_params=pltpu.CompilerParams(
            needs_layout_passes=False, use_tc_tiling_on_sc=True),
    )
    user2d = user.reshape(NW * NCHUNK, CHUNK)
    item2d = item.reshape(NW * NCHUNK, CHUNK)
    uw3 = user_embed_w.T.reshape(PACK, 8, 1000000)
    iw3 = item_embed_w.T.reshape(PACK, 8, 1000000)
    uw_packed = tp(uw3)
    iw_packed = tp(iw3)
    return mf(user2d, item2d, uw_packed, iw_packed)


# speedup vs baseline: 2.1776x; 1.2231x over previous
import jax
import jax.numpy as jnp
from jax import lax
from jax.experimental import pallas as pl
from jax.experimental.pallas import tpu as pltpu
from jax.experimental.pallas import tpu_sc as plsc

BATCH = 16384
D = 32
PACK = 4
W128 = 128
L = 16
NC = 2
NS = 16
NW = NC * NS
BPW = BATCH // NW
CHUNK = 128
NCHUNK = BPW // CHUNK
GPC = CHUNK // L


ROWS_SC = 512
FULL_SC = 1000000 // ROWS_SC
TAIL_ROWS = 1000000 - FULL_SC * ROWS_SC
TAIL_WID = FULL_SC % NW
KMAX = FULL_SC // NW + 1


def _tp_body(t3_hbm, out_hbm, vin0, vin1, vout0, vout1, vtail,
             si0, si1, so0, so1):
    c = lax.axis_index("c")
    s = lax.axis_index("s")
    wid = s * NC + c
    iota = lax.iota(jnp.int32, L)
    zero = jnp.full((L,), 0, jnp.int32)

    def fire_in(sc, vin, si):
        for a in range(PACK):
            pltpu.async_copy(t3_hbm.at[a, :, pl.ds(sc * ROWS_SC, ROWS_SC)],
                             vin.at[pl.ds(a * 8, 8), :], si)

    def drain_in(sc, vin, si):
        for a in range(PACK):
            pltpu.make_async_copy(
                t3_hbm.at[a, :, pl.ds(sc * ROWS_SC, ROWS_SC)],
                vin.at[pl.ds(a * 8, 8), :], si).wait()

    def compute(vin, vout, nq):
        def xblk(x, c2):
            rl = x * L + iota
            qv = lax.shift_right_logical(rl, 2)
            jb = lax.shift_left(jnp.bitwise_and(rl, 3), 5)
            for c in range(D):
                v = vin[c, pl.ds(x * L, L)]
                plsc.store_scatter(vout, [qv, jb + c], v)
            return c2
        lax.fori_loop(0, nq * PACK // L, xblk, 0)

    def step(k, vin, vout, si, so):
        sc = k * NW + wid
        nsc = sc + NW
        nvin, nsi = (vin1, si1) if vin is vin0 else (vin0, si0)

        @pl.when(nsc < FULL_SC)
        def _fire_next():
            fire_in(nsc, nvin, nsi)

        @pl.when(sc < FULL_SC)
        def _do():
            drain_in(sc, vin, si)

            @pl.when(k >= 2)
            def _drain_old_out():
                pltpu.make_async_copy(
                    vout, out_hbm.at[pl.ds(sc * 128, 128), :], so).wait()

            compute(vin, vout, 128)
            pltpu.async_copy(vout, out_hbm.at[pl.ds(sc * 128, 128), :], so)

    fire_in(wid, vin0, si0)

    def body(k, carry):
        @pl.when(k % 2 == 0)
        def _even():
            step(k, vin0, vout0, si0, so0)

        @pl.when(k % 2 == 1)
        def _odd():
            step(k, vin1, vout1, si1, so1)
        return carry

    lax.fori_loop(0, KMAX, body, 0)

    pltpu.make_async_copy(vout0, out_hbm.at[pl.ds(0, 128), :], so0).wait()
    pltpu.make_async_copy(vout1, out_hbm.at[pl.ds(0, 128), :], so1).wait()

    @pl.when(wid == TAIL_WID)
    def _tail():
        toff = (wid - TAIL_WID + 1) * (FULL_SC * ROWS_SC)
        for a in range(PACK):
            pltpu.sync_copy(
                t3_hbm.at[a, :, pl.ds(toff, TAIL_ROWS)],
                vtail.at[pl.ds(a * 8, 8), pl.ds(0, TAIL_ROWS)])
        compute(vtail, vout0, TAIL_ROWS // PACK)
        pltpu.sync_copy(vout0.at[pl.ds(0, TAIL_ROWS // PACK), :],
                        out_hbm.at[pl.ds(FULL_SC * 128, TAIL_ROWS // PACK), :])


def _mf_body(user_hbm, item_hbm, uw_hbm, iw_hbm, out_hbm,
             uidx_v, iidx_v, uq_v, iq_v, urows_v, irows_v, out_v, sem):
    c = lax.axis_index("c")
    s = lax.axis_index("s")
    wid = s * NC + c

    pltpu.sync_copy(user_hbm.at[pl.ds(wid * NCHUNK, NCHUNK)], uidx_v)
    pltpu.sync_copy(item_hbm.at[pl.ds(wid * NCHUNK, NCHUNK)], iidx_v)

    iota = lax.iota(jnp.int32, L)

    def rnd(j, carry):
        def qgrp(i, c2):
            uvec = uidx_v[j, pl.ds(i * L, L)]
            ivec = iidx_v[j, pl.ds(i * L, L)]
            uq_v[pl.ds(i * L, L)] = lax.shift_right_logical(uvec, 2)
            iq_v[pl.ds(i * L, L)] = lax.shift_right_logical(ivec, 2)
            return c2
        lax.fori_loop(0, GPC, qgrp, 0)

        cp1 = pltpu.async_copy(uw_hbm.at[uq_v], urows_v, sem)
        cp2 = pltpu.async_copy(iw_hbm.at[iq_v], irows_v, sem)
        cp1.wait()
        cp2.wait()

        def grp(i, c2):
            rows = i * L + iota
            uvec = uidx_v[j, pl.ds(i * L, L)]
            ivec = iidx_v[j, pl.ds(i * L, L)]
            uoff = lax.shift_left(jnp.bitwise_and(uvec, 3), 5)
            ioff = lax.shift_left(jnp.bitwise_and(ivec, 3), 5)
            acc = jnp.zeros((L,), jnp.float32)
            for d in range(D):
                cu = plsc.load_gather(urows_v, [rows, uoff + d])
                cv = plsc.load_gather(irows_v, [rows, ioff + d])
                acc = acc + cu * cv
            e = jnp.exp(-jnp.abs(acc))
            p = 1.0 / (1.0 + e)
            out_v[pl.ds(j * CHUNK + i * L, L)] = jnp.where(acc >= 0, p, 1.0 - p)
            return c2
        lax.fori_loop(0, GPC, grp, 0)
        return carry

    lax.fori_loop(0, NCHUNK, rnd, 0)
    pltpu.sync_copy(out_v, out_hbm.at[pl.ds(wid * BPW, BPW)])


@jax.jit
def kernel(user, item, user_embed_w, item_embed_w):
    mesh = plsc.VectorSubcoreMesh(core_axis_name="c", subcore_axis_name="s",
                                  num_cores=NC, num_subcores=NS)
    mf = pl.kernel(
        _mf_body,
        out_type=jax.ShapeDtypeStruct((BATCH,), jnp.float32),
        mesh=mesh,
        scratch_types=[
            pltpu.VMEM((NCHUNK, CHUNK), jnp.int32),
            pltpu.VMEM((NCHUNK, CHUNK), jnp.int32),
            pltpu.VMEM((CHUNK,), jnp.int32),
            pltpu.VMEM((CHUNK,), jnp.int32),
            pltpu.VMEM((CHUNK, W128), jnp.float32),
            pltpu.VMEM((CHUNK, W128), jnp.float32),
            pltpu.VMEM((BPW,), jnp.float32),
            pltpu.SemaphoreType.DMA,
        ],
        compiler_params=pltpu.CompilerParams(
            needs_layout_passes=False, use_tc_tiling_on_sc=True),
    )
    tp = pl.kernel(
        _tp_body,
        out_type=jax.ShapeDtypeStruct((1000000 * D // W128, W128),
                                      jnp.float32),
        mesh=mesh,
        scratch_types=[
            pltpu.VMEM((D, ROWS_SC), jnp.float32),
            pltpu.VMEM((D, ROWS_SC), jnp.float32),
            pltpu.VMEM((W128, W128), jnp.float32),
            pltpu.VMEM((W128, W128), jnp.float32),
            pltpu.VMEM((D, W128), jnp.float32),
            pltpu.SemaphoreType.DMA,
            pltpu.SemaphoreType.DMA,
            pltpu.SemaphoreType.DMA,
            pltpu.SemaphoreType.DMA,
        ],
        compiler_params=pltpu.CompilerParams(
            needs_layout_passes=False, use_tc_tiling_on_sc=True),
    )
    user2d = user.reshape(NW * NCHUNK, CHUNK)
    item2d = item.reshape(NW * NCHUNK, CHUNK)
    uw3 = user_embed_w.T.reshape(PACK, 8, 1000000)
    iw3 = item_embed_w.T.reshape(PACK, 8, 1000000)
    uw_packed = tp(uw3)
    iw_packed = tp(iw3)
    return mf(user2d, item2d, uw_packed, iw_packed)


# speedup vs baseline: 3.0602x; 1.4053x over previous
import jax
import jax.numpy as jnp
from jax import lax
from jax.experimental import pallas as pl
from jax.experimental.pallas import tpu as pltpu
from jax.experimental.pallas import tpu_sc as plsc

BATCH = 16384
D = 32
W128 = 128
L = 16
NC = 2
NS = 16
NW = NC * NS
BPW = BATCH // NW
CHUNK = 128
NCHUNK = BPW // CHUNK
GPC = CHUNK // L


def _mf_body(user_hbm, item_hbm, uw_hbm, iw_hbm, out_hbm,
             uidx_v, iidx_v, uq_v, iq_v, urows_v, irows_v, out_v, sem):
    c = lax.axis_index("c")
    s = lax.axis_index("s")
    wid = s * NC + c

    pltpu.sync_copy(user_hbm.at[pl.ds(wid * NCHUNK, NCHUNK)], uidx_v)
    pltpu.sync_copy(item_hbm.at[pl.ds(wid * NCHUNK, NCHUNK)], iidx_v)

    iota = lax.iota(jnp.int32, L)

    def rnd(j, carry):
        def qgrp(i, c2):
            uq_v[pl.ds(i * L, L)] = uidx_v[j, pl.ds(i * L, L)]
            iq_v[pl.ds(i * L, L)] = iidx_v[j, pl.ds(i * L, L)]
            return c2
        lax.fori_loop(0, GPC, qgrp, 0)

        cp1 = pltpu.async_copy(uw_hbm.at[uq_v], urows_v, sem)
        cp2 = pltpu.async_copy(iw_hbm.at[iq_v], irows_v, sem)
        cp1.wait()
        cp2.wait()

        def grp(i, c2):
            rows = i * L + iota
            acc = jnp.zeros((L,), jnp.float32)
            for d in range(D):
                dcol = jnp.full((L,), d, jnp.int32)
                cu = plsc.load_gather(urows_v, [rows, dcol])
                cv = plsc.load_gather(irows_v, [rows, dcol])
                acc = acc + cu * cv
            e = jnp.exp(-jnp.abs(acc))
            p = 1.0 / (1.0 + e)
            out_v[pl.ds(j * CHUNK + i * L, L)] = jnp.where(acc >= 0, p, 1.0 - p)
            return c2
        lax.fori_loop(0, GPC, grp, 0)
        return carry

    lax.fori_loop(0, NCHUNK, rnd, 0)
    pltpu.sync_copy(out_v, out_hbm.at[pl.ds(wid * BPW, BPW)])


@jax.jit
def kernel(user, item, user_embed_w, item_embed_w):
    mesh = plsc.VectorSubcoreMesh(core_axis_name="c", subcore_axis_name="s",
                                  num_cores=NC, num_subcores=NS)
    mf = pl.kernel(
        _mf_body,
        out_type=jax.ShapeDtypeStruct((BATCH,), jnp.float32),
        mesh=mesh,
        scratch_types=[
            pltpu.VMEM((NCHUNK, CHUNK), jnp.int32),
            pltpu.VMEM((NCHUNK, CHUNK), jnp.int32),
            pltpu.VMEM((CHUNK,), jnp.int32),
            pltpu.VMEM((CHUNK,), jnp.int32),
            pltpu.VMEM((CHUNK, W128), jnp.float32),
            pltpu.VMEM((CHUNK, W128), jnp.float32),
            pltpu.VMEM((BPW,), jnp.float32),
            pltpu.SemaphoreType.DMA,
        ],
        compiler_params=pltpu.CompilerParams(
            needs_layout_passes=False, use_tc_tiling_on_sc=True),
    )
    user2d = user.reshape(NW * NCHUNK, CHUNK)
    item2d = item.reshape(NW * NCHUNK, CHUNK)
    uw_pad = jnp.pad(user_embed_w, ((0, 0), (0, W128 - D)))
    iw_pad = jnp.pad(item_embed_w, ((0, 0), (0, W128 - D)))
    return mf(user2d, item2d, uw_pad, iw_pad)


# speedup vs baseline: 3.1148x; 1.0178x over previous
import jax
import jax.numpy as jnp
from jax import lax
from jax.experimental import pallas as pl
from jax.experimental.pallas import tpu as pltpu
from jax.experimental.pallas import tpu_sc as plsc

BATCH = 16384
D = 32
L = 16
NC = 2
NS = 16
NW = NC * NS
BPW = BATCH // NW
CHUNK = 128
NCHUNK = BPW // CHUNK
GROUPS = BPW // L


def _mf_body(user_hbm, item_hbm, uw_hbm, iw_hbm, out_hbm,
             uidx_v, iidx_v, urows_v, irows_v, out_v, sem):
    c = lax.axis_index("c")
    s = lax.axis_index("s")
    wid = s * NC + c

    pltpu.sync_copy(user_hbm.at[pl.ds(wid * NCHUNK, NCHUNK)], uidx_v)
    pltpu.sync_copy(item_hbm.at[pl.ds(wid * NCHUNK, NCHUNK)], iidx_v)

    copies = []
    for j in range(NCHUNK):
        copies.append(pltpu.async_copy(
            uw_hbm.at[uidx_v.at[j]], urows_v.at[pl.ds(j * CHUNK, CHUNK)], sem))
        copies.append(pltpu.async_copy(
            iw_hbm.at[iidx_v.at[j]], irows_v.at[pl.ds(j * CHUNK, CHUNK)], sem))
    for cp in copies:
        cp.wait()

    iota = lax.iota(jnp.int32, L)

    def group(g, carry):
        rows = g * L + iota
        acc = jnp.zeros((L,), jnp.float32)
        for d in range(D):
            dcol = jnp.full((L,), d, jnp.int32)
            cu = plsc.load_gather(urows_v, [rows, dcol])
            cv = plsc.load_gather(irows_v, [rows, dcol])
            acc = acc + cu * cv
        e = jnp.exp(-jnp.abs(acc))
        p = 1.0 / (1.0 + e)
        out_v[pl.ds(g * L, L)] = jnp.where(acc >= 0, p, 1.0 - p)
        return carry

    lax.fori_loop(0, GROUPS, group, 0)
    pltpu.sync_copy(out_v, out_hbm.at[pl.ds(wid * BPW, BPW)])


@jax.jit
def kernel(user, item, user_embed_w, item_embed_w):
    mesh = plsc.VectorSubcoreMesh(core_axis_name="c", subcore_axis_name="s",
                                  num_cores=NC, num_subcores=NS)
    mf = pl.kernel(
        _mf_body,
        out_type=jax.ShapeDtypeStruct((BATCH,), jnp.float32),
        mesh=mesh,
        scratch_types=[
            pltpu.VMEM((NCHUNK, CHUNK), jnp.int32),
            pltpu.VMEM((NCHUNK, CHUNK), jnp.int32),
            pltpu.VMEM((BPW, D), jnp.float32),
            pltpu.VMEM((BPW, D), jnp.float32),
            pltpu.VMEM((BPW,), jnp.float32),
            pltpu.SemaphoreType.DMA,
        ],
        compiler_params=pltpu.CompilerParams(
            needs_layout_passes=False, use_tc_tiling_on_sc=False),
    )
    user2d = user.reshape(NW * NCHUNK, CHUNK)
    item2d = item.reshape(NW * NCHUNK, CHUNK)
    return mf(user2d, item2d, user_embed_w, item_embed_w)
